# speedup vs baseline: 15.1097x; 15.1097x over previous
import functools

import jax
import jax.numpy as jnp
from jax import lax
from jax.experimental import pallas as pl
from jax.experimental.pallas import tpu as pltpu
from jax.experimental.pallas import tpu_sc as plsc

N = 10000
E = 320000
D = 128
H = 128
G = 64

NC = 2
NS = 16
NW = NC * NS

NP = 10240
RPT = NP // NS
EK = 80
EPT = E // NW
NB = EPT // EK

_mesh = plsc.VectorSubcoreMesh(core_axis_name="c", subcore_axis_name="s")


def _deg_body(dst_hbm, zeros_hbm, ones_hbm, deg_out, didx, ones_v, deg_sh):
  cid = lax.axis_index("c")
  sid = lax.axis_index("s")
  wid = cid * NS + sid
  r0 = pl.multiple_of(sid * RPT, 8)
  pltpu.sync_copy(zeros_hbm.at[pl.ds(r0, RPT)], deg_sh.at[pl.ds(r0, RPT)])
  pltpu.sync_copy(ones_hbm, ones_v)
  plsc.subcore_barrier()
  base = wid * EPT

  def body(b, carry):
    off = pl.multiple_of(base + b * EK, 8)
    pltpu.sync_copy(dst_hbm.at[pl.ds(off, EK)], didx)
    pltpu.sync_copy(ones_v, deg_sh.at[didx], add=True)
    return carry

  lax.fori_loop(0, NB, body, 0)
  plsc.subcore_barrier()
  pltpu.sync_copy(deg_sh.at[pl.ds(r0, RPT)], deg_out.at[cid, pl.ds(r0, RPT)])


_sc_deg = functools.partial(
    pl.kernel,
    out_type=jax.ShapeDtypeStruct((NC, NP), jnp.float32),
    mesh=_mesh,
    scratch_types=[
        pltpu.VMEM((EK,), jnp.int32),
        pltpu.VMEM((EK,), jnp.float32),
        pltpu.VMEM_SHARED((NP,), jnp.float32),
    ],
)(_deg_body)


def _wide_body(src_hbm, dst_hbm, g_hbm, zeros2_hbm, acc_out,
               sidx, didx, rows, acc_sh, sem):
  cid = lax.axis_index("c")
  sid = lax.axis_index("s")
  wid = cid * NS + sid
  r0 = pl.multiple_of(sid * RPT, 8)
  pltpu.sync_copy(zeros2_hbm.at[pl.ds(r0, RPT)], acc_sh.at[pl.ds(r0, RPT)])
  plsc.subcore_barrier()
  base = wid * EPT

  def body(b, carry):
    off = pl.multiple_of(base + b * EK, 8)
    pltpu.sync_copy(src_hbm.at[pl.ds(off, EK)], sidx)
    pltpu.sync_copy(dst_hbm.at[pl.ds(off, EK)], didx)
    pltpu.async_copy(g_hbm.at[sidx], rows, sem).wait()
    pltpu.sync_copy(rows, acc_sh.at[didx], add=True)
    return carry

  lax.fori_loop(0, NB, body, 0)
  plsc.subcore_barrier()
  pltpu.sync_copy(acc_sh.at[pl.ds(r0, RPT)], acc_out.at[cid, pl.ds(r0, RPT)])


_sc_wide = functools.partial(
    pl.kernel,
    out_type=jax.ShapeDtypeStruct((NC, NP, D), jnp.float32),
    mesh=_mesh,
    scratch_types=[
        pltpu.VMEM((EK,), jnp.int32),
        pltpu.VMEM((EK,), jnp.int32),
        pltpu.VMEM((EK, D), jnp.float32),
        pltpu.VMEM_SHARED((NP, D), jnp.float32),
        pltpu.SemaphoreType.DMA,
    ],
)(_wide_body)


def _scalar_body(src_hbm, dst_hbm, gz_hbm, zeros_hbm, w_out,
                 sidx, didx, vals, w_sh, sem):
  cid = lax.axis_index("c")
  sid = lax.axis_index("s")
  wid = cid * NS + sid
  r0 = pl.multiple_of(sid * RPT, 8)
  pltpu.sync_copy(zeros_hbm.at[pl.ds(r0, RPT)], w_sh.at[pl.ds(r0, RPT)])
  plsc.subcore_barrier()
  base = wid * EPT

  def body(b, carry):
    off = pl.multiple_of(base + b * EK, 8)
    pltpu.sync_copy(src_hbm.at[pl.ds(off, EK)], sidx)
    pltpu.sync_copy(dst_hbm.at[pl.ds(off, EK)], didx)
    pltpu.async_copy(gz_hbm.at[sidx], vals, sem).wait()
    pltpu.sync_copy(vals, w_sh.at[didx], add=True)
    return carry

  lax.fori_loop(0, NB, body, 0)
  plsc.subcore_barrier()
  pltpu.sync_copy(w_sh.at[pl.ds(r0, RPT)], w_out.at[cid, pl.ds(r0, RPT)])


_sc_scalar = functools.partial(
    pl.kernel,
    out_type=jax.ShapeDtypeStruct((NC, NP), jnp.float32),
    mesh=_mesh,
    scratch_types=[
        pltpu.VMEM((EK,), jnp.int32),
        pltpu.VMEM((EK,), jnp.int32),
        pltpu.VMEM((EK,), jnp.float32),
        pltpu.VMEM_SHARED((NP,), jnp.float32),
        pltpu.SemaphoreType.DMA,
    ],
)(_scalar_body)


_R2 = 2048


def _scale_body(degt_ref, x_ref, dinv_ref, g_ref):
  d = degt_ref[:, 0:1] + degt_ref[:, 1:2] + 1.0
  dinv = lax.rsqrt(d)
  dinv_ref[...] = dinv
  g_ref[...] = x_ref[...] * dinv


def _tc_scale(degt, xp):
  return pl.pallas_call(
      _scale_body,
      grid=(NP // _R2,),
      in_specs=[
          pl.BlockSpec((_R2, NC), lambda i: (i, 0)),
          pl.BlockSpec((_R2, D), lambda i: (i, 0)),
      ],
      out_specs=[
          pl.BlockSpec((_R2, 1), lambda i: (i, 0)),
          pl.BlockSpec((_R2, D), lambda i: (i, 0)),
      ],
      out_shape=[
          jax.ShapeDtypeStruct((NP, 1), jnp.float32),
          jax.ShapeDtypeStruct((NP, D), jnp.float32),
      ],
  )(degt, xp)


_R4 = 2000


def _mlp_body(acc_ref, g_ref, dinv_ref, w1_ref, b1_ref, w2_ref, wc_ref,
              gz_ref):
  q = dinv_ref[...] * (acc_ref[0] + acc_ref[1] + g_ref[...])
  y = jnp.maximum(
      jnp.dot(q, w1_ref[...], preferred_element_type=jnp.float32)
      + b1_ref[...], 0.0)
  u = jnp.dot(w2_ref[...], wc_ref[...], preferred_element_type=jnp.float32)
  z = jnp.dot(y, u, preferred_element_type=jnp.float32)
  gz_ref[...] = dinv_ref[...] * z


def _tc_mlp(acc, g, dinv, W1, b1r, W2, Wc):
  return pl.pallas_call(
      _mlp_body,
      grid=(N // _R4,),
      in_specs=[
          pl.BlockSpec((NC, _R4, D), lambda i: (0, i, 0)),
          pl.BlockSpec((_R4, D), lambda i: (i, 0)),
          pl.BlockSpec((_R4, 1), lambda i: (i, 0)),
          pl.BlockSpec((H, H), lambda i: (0, 0)),
          pl.BlockSpec((1, H), lambda i: (0, 0)),
          pl.BlockSpec((H, H), lambda i: (0, 0)),
          pl.BlockSpec((H, 1), lambda i: (0, 0)),
      ],
      out_specs=pl.BlockSpec((_R4, 1), lambda i: (i, 0)),
      out_shape=jax.ShapeDtypeStruct((N, 1), jnp.float32),
  )(acc, g, dinv, W1, b1r, W2, Wc)


def _pool_body(wt_ref, gz_ref, dinv_ref, batch_ref, b2_ref, wc_ref, bc_ref,
               out_ref, num_ref, cnt_ref):
  i = pl.program_id(0)

  @pl.when(i == 0)
  def _init():
    num_ref[...] = jnp.zeros_like(num_ref)
    cnt_ref[...] = jnp.zeros_like(cnt_ref)

  v = dinv_ref[...] * (wt_ref[:, 0:1] + wt_ref[:, 1:2] + gz_ref[...])
  gids = lax.broadcasted_iota(jnp.int32, (1, G), 1)
  oh = (batch_ref[...] == gids).astype(jnp.float32)
  num_ref[...] += jnp.sum(v * oh, axis=0, keepdims=True)
  cnt_ref[...] += jnp.sum(oh, axis=0, keepdims=True)

  @pl.when(i == pl.num_programs(0) - 1)
  def _fin():
    c0 = jnp.dot(b2_ref[...], wc_ref[...],
                 preferred_element_type=jnp.float32) + bc_ref[...]
    cnt = cnt_ref[...]
    out_ref[...] = (num_ref[...] / jnp.maximum(cnt, 1.0)
                    + jnp.where(cnt > 0.0, c0, bc_ref[...]))


def _tc_pool(wt, gz, dinv, batch2, b2r, Wc, bc2):
  return pl.pallas_call(
      _pool_body,
      grid=(N // _R4,),
      in_specs=[
          pl.BlockSpec((_R4, NC), lambda i: (i, 0)),
          pl.BlockSpec((_R4, 1), lambda i: (i, 0)),
          pl.BlockSpec((_R4, 1), lambda i: (i, 0)),
          pl.BlockSpec((_R4, 1), lambda i: (i, 0)),
          pl.BlockSpec((1, H), lambda i: (0, 0)),
          pl.BlockSpec((H, 1), lambda i: (0, 0)),
          pl.BlockSpec((1, 1), lambda i: (0, 0)),
      ],
      out_specs=pl.BlockSpec((1, G), lambda i: (0, 0)),
      out_shape=jax.ShapeDtypeStruct((1, G), jnp.float32),
      scratch_shapes=[
          pltpu.VMEM((1, G), jnp.float32),
          pltpu.VMEM((1, G), jnp.float32),
      ],
  )(wt, gz, dinv, batch2, b2r, Wc, bc2)


def kernel(x, edge_index, batch, W1, b1, W2, b2, Wc, bc):
  src = edge_index[0]
  dst = edge_index[1]
  xp = jnp.zeros((NP, D), jnp.float32).at[:N].set(x)
  zeros_n = jnp.zeros((NP,), jnp.float32)
  zeros_nd = jnp.zeros((NP, D), jnp.float32)
  ones_k = jnp.ones((EK,), jnp.float32)

  deg_p = _sc_deg(dst, zeros_n, ones_k)
  dinv, g = _tc_scale(deg_p.T, xp)
  acc = _sc_wide(src, dst, g, zeros_nd)
  gz = _tc_mlp(acc, g, dinv, W1, b1.reshape(1, H), W2, Wc)
  gzp = jnp.concatenate([gz[:, 0], jnp.zeros((NP - N,), jnp.float32)])
  w = _sc_scalar(src, dst, gzp, zeros_n)
  out = _tc_pool(w.T, gz, dinv[:N], batch.reshape(N, 1),
                 b2.reshape(1, H), Wc, bc.reshape(1, 1))
  return out.reshape(G, 1)


# speedup vs baseline: 41.3170x; 2.7345x over previous
import functools

import jax
import jax.numpy as jnp
from jax import lax
from jax.experimental import pallas as pl
from jax.experimental.pallas import tpu as pltpu
from jax.experimental.pallas import tpu_sc as plsc

N = 10000
E = 320000
D = 128
H = 128
HD = D // 2
G = 64

NC = 2
NS = 16
NW = NC * NS

NP = 10240
RPT = NP // NS
EK = 80
RING = 6
LAG = 3

EPT_W = E // NS
NB_W = EPT_W // EK
EPT_S = E // NW
NB_S = EPT_S // EK

_mesh = plsc.VectorSubcoreMesh(core_axis_name="c", subcore_axis_name="s")


def _deg_body(dst3, zeros1, ones_hbm, deg_out, didx2, ones_v, deg_sh, ssem):
  cid = lax.axis_index("c")
  sid = lax.axis_index("s")
  wid = cid * NS + sid
  r0 = pl.multiple_of(sid * RPT, 8)
  pltpu.sync_copy(zeros1.at[pl.ds(r0, RPT)], deg_sh.at[pl.ds(r0, RPT)])
  pltpu.sync_copy(ones_hbm, ones_v)
  pltpu.sync_copy(dst3.at[wid], didx2)
  plsc.subcore_barrier()

  def sdesc(b):
    return pltpu.make_async_copy(ones_v, deg_sh.at[didx2.at[b]], ssem)

  def body(b, carry):
    sdesc(b).start(add=True)

    @pl.when(b >= LAG)
    def _():
      sdesc(0).wait()

    return carry

  lax.fori_loop(0, NB_S, body, 0)
  for _ in range(LAG):
    sdesc(0).wait()
  plsc.subcore_barrier()
  pltpu.sync_copy(deg_sh.at[pl.ds(r0, RPT)], deg_out.at[cid, pl.ds(r0, RPT)])


_sc_deg = functools.partial(
    pl.kernel,
    out_type=jax.ShapeDtypeStruct((NC, NP), jnp.float32),
    mesh=_mesh,
    scratch_types=[
        pltpu.VMEM((NB_S, EK), jnp.int32),
        pltpu.VMEM((EK,), jnp.float32),
        pltpu.VMEM_SHARED((NP,), jnp.float32),
        pltpu.SemaphoreType.DMA,
    ],
)(_deg_body)


def _ring_pipeline(nb, gather_start, gdesc0, sdesc, swait0):
  for j in range(LAG):
    gather_start(j)

  def body(b, carry):
    @pl.when(b >= LAG)
    def _():
      swait0()

    @pl.when(b + LAG < nb)
    def _():
      gather_start(b + LAG)

    gdesc0().wait()
    sdesc(b).start(add=True)
    return carry

  lax.fori_loop(0, nb, body, 0)
  for _ in range(LAG):
    swait0()


def _wide_body(src2, dst2, g0, g1, zeros2, acc_out,
               sidx2, didx2, rows, acc_sh, gsem, ssem):
  cid = lax.axis_index("c")
  sid = lax.axis_index("s")
  r0 = pl.multiple_of(sid * RPT, 8)
  pltpu.sync_copy(zeros2.at[pl.ds(r0, RPT)], acc_sh.at[pl.ds(r0, RPT)])
  pltpu.sync_copy(src2.at[sid], sidx2)
  pltpu.sync_copy(dst2.at[sid], didx2)
  plsc.subcore_barrier()

  def sdesc(b):
    return pltpu.make_async_copy(rows.at[b % RING], acc_sh.at[didx2.at[b]],
                                 ssem)

  def run(g_hbm):
    def gather_start(b):
      pltpu.make_async_copy(g_hbm.at[sidx2.at[b]], rows.at[b % RING],
                            gsem).start()

    def gdesc0():
      return pltpu.make_async_copy(g_hbm.at[sidx2.at[0]], rows.at[0], gsem)

    def swait0():
      sdesc(0).wait()

    _ring_pipeline(NB_W, gather_start, gdesc0, sdesc, swait0)

  @pl.when(cid == 0)
  def _():
    run(g0)

  @pl.when(cid == 1)
  def _():
    run(g1)

  plsc.subcore_barrier()
  pltpu.sync_copy(acc_sh.at[pl.ds(r0, RPT)],
                  acc_out.at[cid, pl.ds(r0, RPT)])


_sc_wide = functools.partial(
    pl.kernel,
    out_type=jax.ShapeDtypeStruct((NC, NP, HD), jnp.float32),
    mesh=_mesh,
    compiler_params=pltpu.CompilerParams(use_tc_tiling_on_sc=False),
    scratch_types=[
        pltpu.VMEM((NB_W, EK), jnp.int32),
        pltpu.VMEM((NB_W, EK), jnp.int32),
        pltpu.VMEM((RING, EK, HD), jnp.float32),
        pltpu.VMEM_SHARED((NP, HD), jnp.float32),
        pltpu.SemaphoreType.DMA,
        pltpu.SemaphoreType.DMA,
    ],
)(_wide_body)


def _scalar_body(src3, dst3, gz_hbm, zeros1, w_out,
                 sidx2, didx2, vals, w_sh, gsem, ssem):
  cid = lax.axis_index("c")
  sid = lax.axis_index("s")
  wid = cid * NS + sid
  r0 = pl.multiple_of(sid * RPT, 8)
  pltpu.sync_copy(zeros1.at[pl.ds(r0, RPT)], w_sh.at[pl.ds(r0, RPT)])
  pltpu.sync_copy(src3.at[wid], sidx2)
  pltpu.sync_copy(dst3.at[wid], didx2)
  plsc.subcore_barrier()

  def gather_start(b):
    pltpu.make_async_copy(gz_hbm.at[sidx2.at[b]], vals.at[b % RING],
                          gsem).start()

  def gdesc0():
    return pltpu.make_async_copy(gz_hbm.at[sidx2.at[0]], vals.at[0], gsem)

  def sdesc(b):
    return pltpu.make_async_copy(vals.at[b % RING], w_sh.at[didx2.at[b]],
                                 ssem)

  def swait0():
    sdesc(0).wait()

  _ring_pipeline(NB_S, gather_start, gdesc0, sdesc, swait0)
  plsc.subcore_barrier()
  pltpu.sync_copy(w_sh.at[pl.ds(r0, RPT)], w_out.at[cid, pl.ds(r0, RPT)])


_sc_scalar = functools.partial(
    pl.kernel,
    out_type=jax.ShapeDtypeStruct((NC, NP), jnp.float32),
    mesh=_mesh,
    scratch_types=[
        pltpu.VMEM((NB_S, EK), jnp.int32),
        pltpu.VMEM((NB_S, EK), jnp.int32),
        pltpu.VMEM((RING, EK), jnp.float32),
        pltpu.VMEM_SHARED((NP,), jnp.float32),
        pltpu.SemaphoreType.DMA,
        pltpu.SemaphoreType.DMA,
    ],
)(_scalar_body)


_RB = 2000


def _scale_body(degt_ref, x_ref, dinv_ref, g0_ref, g1_ref):
  d = degt_ref[:, 0:1] + degt_ref[:, 1:2] + 1.0
  dinv = lax.rsqrt(d)
  dinv_ref[...] = dinv
  g0_ref[...] = x_ref[:, :HD] * dinv
  g1_ref[...] = x_ref[:, HD:] * dinv


def _tc_scale(degt, x):
  return pl.pallas_call(
      _scale_body,
      grid=(N // _RB,),
      in_specs=[
          pl.BlockSpec((_RB, NC), lambda i: (i, 0)),
          pl.BlockSpec((_RB, D), lambda i: (i, 0)),
      ],
      out_specs=[
          pl.BlockSpec((_RB, 1), lambda i: (i, 0)),
          pl.BlockSpec((_RB, HD), lambda i: (i, 0)),
          pl.BlockSpec((_RB, HD), lambda i: (i, 0)),
      ],
      out_shape=[
          jax.ShapeDtypeStruct((NP, 1), jnp.float32),
          jax.ShapeDtypeStruct((NP, HD), jnp.float32),
          jax.ShapeDtypeStruct((NP, HD), jnp.float32),
      ],
  )(degt, x)


def _mlp_body(acc_ref, g0_ref, g1_ref, dinv_ref, w1_ref, b1_ref, w2_ref,
              wc_ref, gz_ref):
  q = dinv_ref[...] * jnp.concatenate(
      [acc_ref[0] + g0_ref[...], acc_ref[1] + g1_ref[...]], axis=1)
  y = jnp.maximum(
      jnp.dot(q, w1_ref[...], preferred_element_type=jnp.float32)
      + b1_ref[...], 0.0)
  u = jnp.dot(w2_ref[...], wc_ref[...], preferred_element_type=jnp.float32)
  z = jnp.dot(y, u, preferred_element_type=jnp.float32)
  gz_ref[...] = dinv_ref[...] * z


def _tc_mlp(acc, g0, g1, dinv, W1, b1r, W2, Wc):
  return pl.pallas_call(
      _mlp_body,
      grid=(N // _RB,),
      in_specs=[
          pl.BlockSpec((NC, _RB, HD), lambda i: (0, i, 0)),
          pl.BlockSpec((_RB, HD), lambda i: (i, 0)),
          pl.BlockSpec((_RB, HD), lambda i: (i, 0)),
          pl.BlockSpec((_RB, 1), lambda i: (i, 0)),
          pl.BlockSpec((H, H), lambda i: (0, 0)),
          pl.BlockSpec((1, H), lambda i: (0, 0)),
          pl.BlockSpec((H, H), lambda i: (0, 0)),
          pl.BlockSpec((H, 1), lambda i: (0, 0)),
      ],
      out_specs=pl.BlockSpec((_RB, 1), lambda i: (i, 0)),
      out_shape=jax.ShapeDtypeStruct((NP, 1), jnp.float32),
  )(acc, g0, g1, dinv, W1, b1r, W2, Wc)


def _pool_body(wt_ref, gz_ref, dinv_ref, batch_ref, b2_ref, wc_ref, bc_ref,
               out_ref, num_ref, cnt_ref):
  i = pl.program_id(0)

  @pl.when(i == 0)
  def _init():
    num_ref[...] = jnp.zeros_like(num_ref)
    cnt_ref[...] = jnp.zeros_like(cnt_ref)

  v = dinv_ref[...] * (wt_ref[:, 0:1] + wt_ref[:, 1:2] + gz_ref[...])
  gids = lax.broadcasted_iota(jnp.int32, (1, G), 1)
  oh = (batch_ref[...] == gids).astype(jnp.float32)
  num_ref[...] += jnp.sum(v * oh, axis=0, keepdims=True)
  cnt_ref[...] += jnp.sum(oh, axis=0, keepdims=True)

  @pl.when(i == pl.num_programs(0) - 1)
  def _fin():
    c0 = jnp.dot(b2_ref[...], wc_ref[...],
                 preferred_element_type=jnp.float32) + bc_ref[...]
    cnt = cnt_ref[...]
    out_ref[...] = (num_ref[...] / jnp.maximum(cnt, 1.0)
                    + jnp.where(cnt > 0.0, c0, bc_ref[...]))


def _tc_pool(wt, gz, dinv, batch2, b2r, Wc, bc2):
  return pl.pallas_call(
      _pool_body,
      grid=(N // _RB,),
      in_specs=[
          pl.BlockSpec((_RB, NC), lambda i: (i, 0)),
          pl.BlockSpec((_RB, 1), lambda i: (i, 0)),
          pl.BlockSpec((_RB, 1), lambda i: (i, 0)),
          pl.BlockSpec((_RB, 1), lambda i: (i, 0)),
          pl.BlockSpec((1, H), lambda i: (0, 0)),
          pl.BlockSpec((H, 1), lambda i: (0, 0)),
          pl.BlockSpec((1, 1), lambda i: (0, 0)),
      ],
      out_specs=pl.BlockSpec((1, G), lambda i: (0, 0)),
      out_shape=jax.ShapeDtypeStruct((1, G), jnp.float32),
      scratch_shapes=[
          pltpu.VMEM((1, G), jnp.float32),
          pltpu.VMEM((1, G), jnp.float32),
      ],
  )(wt, gz, dinv, batch2, b2r, Wc, bc2)


def kernel(x, edge_index, batch, W1, b1, W2, b2, Wc, bc):
  src = edge_index[0]
  dst = edge_index[1]
  src2 = src.reshape(NS, NB_W, EK)
  dst2 = dst.reshape(NS, NB_W, EK)
  src3 = src.reshape(NW, NB_S, EK)
  dst3 = dst.reshape(NW, NB_S, EK)
  zeros_n = jnp.zeros((NP,), jnp.float32)
  zeros_nh = jnp.zeros((NP, HD), jnp.float32)
  ones_k = jnp.ones((EK,), jnp.float32)

  deg_p = _sc_deg(dst3, zeros_n, ones_k)
  dinv, g0, g1 = _tc_scale(deg_p.T, x)
  acc = _sc_wide(src2, dst2, g0, g1, zeros_nh)
  gz = _tc_mlp(acc, g0, g1, dinv, W1, b1.reshape(1, H), W2, Wc)
  w = _sc_scalar(src3, dst3, gz.reshape(NP), zeros_n)
  out = _tc_pool(w.T, gz, dinv, batch.reshape(N, 1),
                 b2.reshape(1, H), Wc, bc.reshape(1, 1))
  return out.reshape(G, 1)


# speedup vs baseline: 49.2290x; 1.1915x over previous
import functools

import jax
import jax.numpy as jnp
from jax import lax
from jax.experimental import pallas as pl
from jax.experimental.pallas import tpu as pltpu
from jax.experimental.pallas import tpu_sc as plsc

N = 10000
E = 320000
D = 128
H = 128
HD = D // 2
G = 64
GB = 128

NC = 2
NS = 16
NW = NC * NS

NP = 10240
RPT = NP // NS
EK = 80
RING = 6
LAG = 3

EPT_W = E // NS
NB_W = EPT_W // EK
EPT_S = E // NW
NB_S = EPT_S // EK
NPB = NP // NW // EK
NPS = NP // NS // EK

_mesh = plsc.VectorSubcoreMesh(core_axis_name="c", subcore_axis_name="s")


def _deg_body(dst3, batch4, zeros1, ones_hbm, deg_out, cnt_out,
              didx2, bidx2, ones_v, deg_sh, cnt_sh, ssem):
  cid = lax.axis_index("c")
  sid = lax.axis_index("s")
  wid = cid * NS + sid
  r0 = pl.multiple_of(sid * RPT, 8)
  pltpu.sync_copy(zeros1.at[pl.ds(r0, RPT)], deg_sh.at[pl.ds(r0, RPT)])

  @pl.when(sid == 0)
  def _():
    pltpu.sync_copy(zeros1.at[pl.ds(0, GB)], cnt_sh)

  pltpu.sync_copy(ones_hbm, ones_v)
  pltpu.sync_copy(dst3.at[wid], didx2)
  pltpu.sync_copy(batch4.at[wid], bidx2)
  plsc.subcore_barrier()

  def sdesc(b):
    return pltpu.make_async_copy(ones_v, deg_sh.at[didx2.at[b]], ssem)

  def body(b, carry):
    sdesc(b).start(add=True)

    @pl.when(b >= LAG)
    def _():
      sdesc(0).wait()

    return carry

  lax.fori_loop(0, NB_S, body, 0)
  for _ in range(LAG):
    sdesc(0).wait()
  for q in range(NPB):
    pltpu.make_async_copy(ones_v, cnt_sh.at[bidx2.at[q]], ssem).start(add=True)
  for q in range(NPB):
    pltpu.make_async_copy(ones_v, cnt_sh.at[bidx2.at[0]], ssem).wait()
  plsc.subcore_barrier()
  pltpu.sync_copy(deg_sh.at[pl.ds(r0, RPT)], deg_out.at[cid, pl.ds(r0, RPT)])

  @pl.when(sid == 0)
  def _():
    pltpu.sync_copy(cnt_sh, cnt_out.at[cid])


_sc_deg = functools.partial(
    pl.kernel,
    out_type=[
        jax.ShapeDtypeStruct((NC, NP), jnp.float32),
        jax.ShapeDtypeStruct((NC, GB), jnp.float32),
    ],
    mesh=_mesh,
    scratch_types=[
        pltpu.VMEM((NB_S, EK), jnp.int32),
        pltpu.VMEM((NPB, EK), jnp.int32),
        pltpu.VMEM((EK,), jnp.float32),
        pltpu.VMEM_SHARED((NP,), jnp.float32),
        pltpu.VMEM_SHARED((GB,), jnp.float32),
        pltpu.SemaphoreType.DMA,
    ],
)(_deg_body)


def _wide_body(src2, dst2, g0, g1, zeros2, acc_out,
               sidx2, didx2, rows, acc_sh, gsem, ssem):
  cid = lax.axis_index("c")
  sid = lax.axis_index("s")
  r0 = pl.multiple_of(sid * RPT, 8)
  pltpu.sync_copy(zeros2.at[pl.ds(r0, RPT)], acc_sh.at[pl.ds(r0, RPT)])
  pltpu.sync_copy(src2.at[sid], sidx2)
  pltpu.sync_copy(dst2.at[sid], didx2)
  plsc.subcore_barrier()

  def sdesc(b):
    return pltpu.make_async_copy(rows.at[b % RING], acc_sh.at[didx2.at[b]],
                                 ssem)

  def run(g_hbm):
    def gstart(b):
      pltpu.make_async_copy(g_hbm.at[sidx2.at[b]], rows.at[b % RING],
                            gsem).start()

    for j in range(LAG):
      gstart(j)

    def body(b, carry):
      @pl.when(b >= LAG)
      def _():
        sdesc(0).wait()

      @pl.when(b + LAG < NB_W)
      def _():
        gstart(b + LAG)

      pltpu.make_async_copy(g_hbm.at[sidx2.at[0]], rows.at[0], gsem).wait()
      sdesc(b).start(add=True)
      return carry

    lax.fori_loop(0, NB_W, body, 0)
    for _ in range(LAG):
      sdesc(0).wait()

  @pl.when(cid == 0)
  def _():
    run(g0)

  @pl.when(cid == 1)
  def _():
    run(g1)

  plsc.subcore_barrier()
  pltpu.sync_copy(acc_sh.at[pl.ds(r0, RPT)], acc_out.at[cid, pl.ds(r0, RPT)])


_sc_wide = functools.partial(
    pl.kernel,
    out_type=jax.ShapeDtypeStruct((NC, NP, HD), jnp.float32),
    mesh=_mesh,
    compiler_params=pltpu.CompilerParams(use_tc_tiling_on_sc=False),
    scratch_types=[
        pltpu.VMEM((NB_W, EK), jnp.int32),
        pltpu.VMEM((NB_W, EK), jnp.int32),
        pltpu.VMEM((RING, EK, HD), jnp.float32),
        pltpu.VMEM_SHARED((NP, HD), jnp.float32),
        pltpu.SemaphoreType.DMA,
        pltpu.SemaphoreType.DMA,
    ],
)(_wide_body)


def _scalar_pool_body(src_f, dst3, gz_hbm, dinv_hbm, batch3, zeros1, num_out,
                      gzrep, sidx_f, didx2, bidx2, stage, wv, dv, tv,
                      w_sh, bins_sh, ssem):
  cid = lax.axis_index("c")
  sid = lax.axis_index("s")
  wid = cid * NS + sid
  r0 = pl.multiple_of(sid * RPT, 8)
  pltpu.sync_copy(zeros1.at[pl.ds(r0, RPT)], w_sh.at[pl.ds(r0, RPT)])

  @pl.when(sid == 0)
  def _():
    pltpu.sync_copy(zeros1.at[pl.ds(0, GB)], bins_sh)

  pltpu.sync_copy(gz_hbm, gzrep)
  pltpu.sync_copy(src_f.at[wid], sidx_f)
  pltpu.sync_copy(dst3.at[wid], didx2)
  pltpu.sync_copy(batch3.at[sid], bidx2)
  pltpu.sync_copy(dinv_hbm.at[pl.ds(r0, RPT)], dv)
  plsc.subcore_barrier()

  def sdesc(b, slot):
    return pltpu.make_async_copy(stage.at[pl.ds(slot * EK, EK)],
                                 w_sh.at[didx2.at[b]], ssem)

  def body(b, carry):
    slot = b % RING

    @pl.when(b >= RING)
    def _():
      sdesc(0, 0).wait()

    for j in range(EK // 16):
      idxv = sidx_f[pl.ds(b * EK + j * 16, 16)]
      vals = plsc.load_gather(gzrep, [idxv])
      stage[pl.ds(slot * EK + j * 16, 16)] = vals
    sdesc(b, slot).start(add=True)
    return carry

  lax.fori_loop(0, NB_S, body, 0)
  for _ in range(RING):
    sdesc(0, 0).wait()
  plsc.subcore_barrier()

  pltpu.sync_copy(w_sh.at[pl.ds(r0, RPT)], wv)

  @pl.when(cid == 0)
  def _():
    for i in range(RPT // 16):
      o = i * 16
      wv[pl.ds(o, 16)] = wv[pl.ds(o, 16)] + gzrep[pl.ds(r0 + o, 16)]

  for i in range(RPT // 16):
    o = i * 16
    tv[pl.ds(o, 16)] = dv[pl.ds(o, 16)] * wv[pl.ds(o, 16)]
  for q in range(NPS):
    pltpu.make_async_copy(tv.at[pl.ds(q * EK, EK)],
                          bins_sh.at[bidx2.at[q]], ssem).start(add=True)
  for q in range(NPS):
    pltpu.make_async_copy(tv.at[pl.ds(0, EK)],
                          bins_sh.at[bidx2.at[0]], ssem).wait()
  plsc.subcore_barrier()

  @pl.when(sid == 0)
  def _():
    pltpu.sync_copy(bins_sh, num_out.at[cid])


_sc_scalar_pool = functools.partial(
    pl.kernel,
    out_type=jax.ShapeDtypeStruct((NC, GB), jnp.float32),
    mesh=_mesh,
    compiler_params=pltpu.CompilerParams(needs_layout_passes=False),
    scratch_types=[
        pltpu.VMEM((NP,), jnp.float32),
        pltpu.VMEM((EPT_S,), jnp.int32),
        pltpu.VMEM((NB_S, EK), jnp.int32),
        pltpu.VMEM((NPS, EK), jnp.int32),
        pltpu.VMEM((RING * EK,), jnp.float32),
        pltpu.VMEM((RPT,), jnp.float32),
        pltpu.VMEM((RPT,), jnp.float32),
        pltpu.VMEM((RPT,), jnp.float32),
        pltpu.VMEM_SHARED((NP,), jnp.float32),
        pltpu.VMEM_SHARED((GB,), jnp.float32),
        pltpu.SemaphoreType.DMA,
    ],
)(_scalar_pool_body)


_RB = 2000


def _scale_body(degt_ref, x_ref, dinv_ref, g0_ref, g1_ref):
  d = degt_ref[:, 0:1] + degt_ref[:, 1:2] + 1.0
  dinv = lax.rsqrt(d)
  dinv_ref[...] = dinv
  g0_ref[...] = x_ref[:, :HD] * dinv
  g1_ref[...] = x_ref[:, HD:] * dinv


def _tc_scale(degt, x):
  return pl.pallas_call(
      _scale_body,
      grid=(N // _RB,),
      in_specs=[
          pl.BlockSpec((_RB, NC), lambda i: (i, 0)),
          pl.BlockSpec((_RB, D), lambda i: (i, 0)),
      ],
      out_specs=[
          pl.BlockSpec((_RB, 1), lambda i: (i, 0)),
          pl.BlockSpec((_RB, HD), lambda i: (i, 0)),
          pl.BlockSpec((_RB, HD), lambda i: (i, 0)),
      ],
      out_shape=[
          jax.ShapeDtypeStruct((NP, 1), jnp.float32),
          jax.ShapeDtypeStruct((NP, HD), jnp.float32),
          jax.ShapeDtypeStruct((NP, HD), jnp.float32),
      ],
  )(degt, x)


def _mlp_body(acc_ref, g0_ref, g1_ref, dinv_ref, w1_ref, b1_ref, w2_ref,
              wc_ref, b2_ref, bc_ref, gz_ref, c0_ref):
  q = dinv_ref[...] * jnp.concatenate(
      [acc_ref[0] + g0_ref[...], acc_ref[1] + g1_ref[...]], axis=1)
  y = jnp.maximum(
      jnp.dot(q, w1_ref[...], preferred_element_type=jnp.float32)
      + b1_ref[...], 0.0)
  u = jnp.dot(w2_ref[...], wc_ref[...], preferred_element_type=jnp.float32)
  z = jnp.dot(y, u, preferred_element_type=jnp.float32)
  gz_ref[...] = dinv_ref[...] * z
  c0_ref[...] = jnp.dot(b2_ref[...], wc_ref[...],
                        preferred_element_type=jnp.float32) + bc_ref[...]


def _tc_mlp(acc, g0, g1, dinv, W1, b1r, W2, Wc, b2r, bc2):
  return pl.pallas_call(
      _mlp_body,
      grid=(N // _RB,),
      in_specs=[
          pl.BlockSpec((NC, _RB, HD), lambda i: (0, i, 0)),
          pl.BlockSpec((_RB, HD), lambda i: (i, 0)),
          pl.BlockSpec((_RB, HD), lambda i: (i, 0)),
          pl.BlockSpec((_RB, 1), lambda i: (i, 0)),
          pl.BlockSpec((H, H), lambda i: (0, 0)),
          pl.BlockSpec((1, H), lambda i: (0, 0)),
          pl.BlockSpec((H, H), lambda i: (0, 0)),
          pl.BlockSpec((H, 1), lambda i: (0, 0)),
          pl.BlockSpec((1, H), lambda i: (0, 0)),
          pl.BlockSpec((1, 1), lambda i: (0, 0)),
      ],
      out_specs=[
          pl.BlockSpec((_RB, 1), lambda i: (i, 0)),
          pl.BlockSpec((1, 1), lambda i: (0, 0)),
      ],
      out_shape=[
          jax.ShapeDtypeStruct((NP, 1), jnp.float32),
          jax.ShapeDtypeStruct((1, 1), jnp.float32),
      ],
  )(acc, g0, g1, dinv, W1, b1r, W2, Wc, b2r, bc2)


def kernel(x, edge_index, batch, W1, b1, W2, b2, Wc, bc):
  src = edge_index[0]
  dst = edge_index[1]
  src2 = src.reshape(NS, NB_W, EK)
  dst2 = dst.reshape(NS, NB_W, EK)
  src_f = src.reshape(NW, EPT_S)
  dst3 = dst.reshape(NW, NB_S, EK)
  batch_pad = jnp.concatenate([batch, jnp.full((NP - N,), G, jnp.int32)])
  batch4 = batch_pad.reshape(NW, NPB, EK)
  batch3 = batch_pad.reshape(NS, NPS, EK)
  zeros_n = jnp.zeros((NP,), jnp.float32)
  zeros_nh = jnp.zeros((NP, HD), jnp.float32)
  ones_k = jnp.ones((EK,), jnp.float32)

  deg_p, cnt_p = _sc_deg(dst3, batch4, zeros_n, ones_k)
  dinv, g0, g1 = _tc_scale(deg_p.T, x)
  acc = _sc_wide(src2, dst2, g0, g1, zeros_nh)
  gz, c0 = _tc_mlp(acc, g0, g1, dinv, W1, b1.reshape(1, H), W2, Wc,
                   b2.reshape(1, H), bc.reshape(1, 1))
  num_p = _sc_scalar_pool(src_f, dst3, gz.reshape(NP), dinv.reshape(NP),
                          batch3, zeros_n)
  num = num_p[0, :G] + num_p[1, :G]
  cnt = cnt_p[0, :G] + cnt_p[1, :G]
  out = num / jnp.maximum(cnt, 1.0) + jnp.where(cnt > 0, c0[0, 0], bc[0])
  return out.reshape(G, 1)


# speedup vs baseline: 49.9060x; 1.0138x over previous
import functools

import jax
import jax.numpy as jnp
from jax import lax
from jax.experimental import pallas as pl
from jax.experimental.pallas import tpu as pltpu
from jax.experimental.pallas import tpu_sc as plsc

N = 10000
E = 320000
D = 128
H = 128
HD = D // 2
G = 64
GB = 128

NC = 2
NS = 16
NW = NC * NS

NP = 10240
RPT = NP // NS
EK = 80
RING = 6
LAG = 3

EPT_W = E // NS
NB_W = EPT_W // EK
EPT_S = E // NW
NB_S = EPT_S // EK
NPB = NP // NW // EK
NPS = NP // NS // EK

_mesh = plsc.VectorSubcoreMesh(core_axis_name="c", subcore_axis_name="s")


def _deg_body(dst3, batch3, zeros1, ones_hbm, deg_out, cnt_out,
              didx2, bidx2, ones_v, deg_sh, cnt_sh, ssem):
  cid = lax.axis_index("c")
  sid = lax.axis_index("s")
  wid = cid * NS + sid
  r0 = pl.multiple_of(sid * RPT, 8)
  pltpu.sync_copy(zeros1.at[pl.ds(r0, RPT)], deg_sh.at[pl.ds(r0, RPT)])

  @pl.when(sid == 0)
  def _():
    pltpu.sync_copy(zeros1.at[pl.ds(0, GB)], cnt_sh)

  pltpu.sync_copy(ones_hbm, ones_v)
  pltpu.sync_copy(dst3.at[wid], didx2)
  pltpu.sync_copy(batch3.at[sid], bidx2)
  plsc.subcore_barrier()

  def sdesc(b):
    return pltpu.make_async_copy(ones_v, deg_sh.at[didx2.at[b]], ssem)

  def body(b, carry):
    sdesc(b).start(add=True)

    @pl.when(b >= LAG)
    def _():
      sdesc(0).wait()

    return carry

  lax.fori_loop(0, NB_S, body, 0)
  for _ in range(LAG):
    sdesc(0).wait()

  @pl.when(cid == 0)
  def _():
    for q in range(NPS):
      pltpu.make_async_copy(ones_v, cnt_sh.at[bidx2.at[q]],
                            ssem).start(add=True)
    for q in range(NPS):
      pltpu.make_async_copy(ones_v, cnt_sh.at[bidx2.at[0]], ssem).wait()

  plsc.subcore_barrier()
  pltpu.sync_copy(deg_sh.at[pl.ds(r0, RPT)], deg_out.at[cid, pl.ds(r0, RPT)])

  @pl.when(sid == 0)
  def _():
    pltpu.sync_copy(cnt_sh, cnt_out.at[cid])


_sc_deg = functools.partial(
    pl.kernel,
    out_type=[
        jax.ShapeDtypeStruct((NC, NP), jnp.float32),
        jax.ShapeDtypeStruct((NC, GB), jnp.float32),
    ],
    mesh=_mesh,
    compiler_params=pltpu.CompilerParams(use_tc_tiling_on_sc=False),
    scratch_types=[
        pltpu.VMEM((NB_S, EK), jnp.int32),
        pltpu.VMEM((NPS, EK), jnp.int32),
        pltpu.VMEM((EK,), jnp.float32),
        pltpu.VMEM_SHARED((NP,), jnp.float32),
        pltpu.VMEM_SHARED((GB,), jnp.float32),
        pltpu.SemaphoreType.DMA,
    ],
)(_deg_body)


def _wide_body(src2, dst2, g0, g1, zeros2, acc_out,
               sidx2, didx2, rows, acc_sh, gsem, ssem):
  cid = lax.axis_index("c")
  sid = lax.axis_index("s")
  r0 = pl.multiple_of(sid * RPT, 8)
  pltpu.sync_copy(zeros2.at[pl.ds(r0, RPT)], acc_sh.at[pl.ds(r0, RPT)])
  pltpu.sync_copy(src2.at[sid], sidx2)
  pltpu.sync_copy(dst2.at[sid], didx2)
  plsc.subcore_barrier()

  def sdesc(b):
    return pltpu.make_async_copy(rows.at[b % RING], acc_sh.at[didx2.at[b]],
                                 ssem)

  def run(g_hbm):
    def gstart(b):
      pltpu.make_async_copy(g_hbm.at[sidx2.at[b]], rows.at[b % RING],
                            gsem).start()

    for j in range(LAG):
      gstart(j)

    def body(b, carry):
      @pl.when(b >= LAG)
      def _():
        sdesc(0).wait()

      @pl.when(b + LAG < NB_W)
      def _():
        gstart(b + LAG)

      pltpu.make_async_copy(g_hbm.at[sidx2.at[0]], rows.at[0], gsem).wait()
      sdesc(b).start(add=True)
      return carry

    lax.fori_loop(0, NB_W, body, 0)
    for _ in range(LAG):
      sdesc(0).wait()

  @pl.when(cid == 0)
  def _():
    run(g0)

  @pl.when(cid == 1)
  def _():
    run(g1)

  plsc.subcore_barrier()
  pltpu.sync_copy(acc_sh.at[pl.ds(r0, RPT)], acc_out.at[cid, pl.ds(r0, RPT)])


_sc_wide = functools.partial(
    pl.kernel,
    out_type=jax.ShapeDtypeStruct((NC, NP, HD), jnp.float32),
    mesh=_mesh,
    compiler_params=pltpu.CompilerParams(use_tc_tiling_on_sc=False),
    scratch_types=[
        pltpu.VMEM((NB_W, EK), jnp.int32),
        pltpu.VMEM((NB_W, EK), jnp.int32),
        pltpu.VMEM((RING, EK, HD), jnp.float32),
        pltpu.VMEM_SHARED((NP, HD), jnp.float32),
        pltpu.SemaphoreType.DMA,
        pltpu.SemaphoreType.DMA,
    ],
)(_wide_body)


def _scalar_pool_body(src_f, dst3, gz_hbm, dinv_hbm, batch3, zeros1, num_out,
                      gzrep, sidx_f, didx2, bidx2, stage, wv, dv, tv,
                      w_sh, bins_sh, ssem):
  cid = lax.axis_index("c")
  sid = lax.axis_index("s")
  wid = cid * NS + sid
  r0 = pl.multiple_of(sid * RPT, 8)
  pltpu.sync_copy(zeros1.at[pl.ds(r0, RPT)], w_sh.at[pl.ds(r0, RPT)])

  @pl.when(sid == 0)
  def _():
    pltpu.sync_copy(zeros1.at[pl.ds(0, GB)], bins_sh)

  pltpu.sync_copy(gz_hbm, gzrep)
  pltpu.sync_copy(src_f.at[wid], sidx_f)
  pltpu.sync_copy(dst3.at[wid], didx2)
  pltpu.sync_copy(batch3.at[sid], bidx2)
  pltpu.sync_copy(dinv_hbm.at[pl.ds(r0, RPT)], dv)
  plsc.subcore_barrier()

  def sdesc(b, slot):
    return pltpu.make_async_copy(stage.at[pl.ds(slot * EK, EK)],
                                 w_sh.at[didx2.at[b]], ssem)

  def body(b, carry):
    slot = b % RING

    @pl.when(b >= RING)
    def _():
      sdesc(0, 0).wait()

    for j in range(EK // 16):
      idxv = sidx_f[pl.ds(b * EK + j * 16, 16)]
      vals = plsc.load_gather(gzrep, [idxv])
      stage[pl.ds(slot * EK + j * 16, 16)] = vals
    sdesc(b, slot).start(add=True)
    return carry

  lax.fori_loop(0, NB_S, body, 0)
  for _ in range(RING):
    sdesc(0, 0).wait()
  plsc.subcore_barrier()

  pltpu.sync_copy(w_sh.at[pl.ds(r0, RPT)], wv)

  @pl.when(cid == 0)
  def _():
    for i in range(RPT // 16):
      o = i * 16
      wv[pl.ds(o, 16)] = wv[pl.ds(o, 16)] + gzrep[pl.ds(r0 + o, 16)]

  for i in range(RPT // 16):
    o = i * 16
    tv[pl.ds(o, 16)] = dv[pl.ds(o, 16)] * wv[pl.ds(o, 16)]
  for q in range(NPS):
    pltpu.make_async_copy(tv.at[pl.ds(q * EK, EK)],
                          bins_sh.at[bidx2.at[q]], ssem).start(add=True)
  for q in range(NPS):
    pltpu.make_async_copy(tv.at[pl.ds(0, EK)],
                          bins_sh.at[bidx2.at[0]], ssem).wait()
  plsc.subcore_barrier()

  @pl.when(sid == 0)
  def _():
    pltpu.sync_copy(bins_sh, num_out.at[cid])


_sc_scalar_pool = functools.partial(
    pl.kernel,
    out_type=jax.ShapeDtypeStruct((NC, GB), jnp.float32),
    mesh=_mesh,
    compiler_params=pltpu.CompilerParams(needs_layout_passes=False,
                                         use_tc_tiling_on_sc=False),
    scratch_types=[
        pltpu.VMEM((NP,), jnp.float32),
        pltpu.VMEM((EPT_S,), jnp.int32),
        pltpu.VMEM((NB_S, EK), jnp.int32),
        pltpu.VMEM((NPS, EK), jnp.int32),
        pltpu.VMEM((RING * EK,), jnp.float32),
        pltpu.VMEM((RPT,), jnp.float32),
        pltpu.VMEM((RPT,), jnp.float32),
        pltpu.VMEM((RPT,), jnp.float32),
        pltpu.VMEM_SHARED((NP,), jnp.float32),
        pltpu.VMEM_SHARED((GB,), jnp.float32),
        pltpu.SemaphoreType.DMA,
    ],
)(_scalar_pool_body)


_RB = 2000


def _scale_body(degt_ref, x_ref, dinv_ref, g0_ref, g1_ref):
  d = degt_ref[:, 0:1] + degt_ref[:, 1:2] + 1.0
  dinv = lax.rsqrt(d)
  dinv_ref[...] = dinv
  g0_ref[...] = x_ref[:, :HD] * dinv
  g1_ref[...] = x_ref[:, HD:] * dinv


def _tc_scale(degt, x):
  return pl.pallas_call(
      _scale_body,
      grid=(N // _RB,),
      in_specs=[
          pl.BlockSpec((_RB, NC), lambda i: (i, 0)),
          pl.BlockSpec((_RB, D), lambda i: (i, 0)),
      ],
      out_specs=[
          pl.BlockSpec((_RB, 1), lambda i: (i, 0)),
          pl.BlockSpec((_RB, HD), lambda i: (i, 0)),
          pl.BlockSpec((_RB, HD), lambda i: (i, 0)),
      ],
      out_shape=[
          jax.ShapeDtypeStruct((NP, 1), jnp.float32),
          jax.ShapeDtypeStruct((NP, HD), jnp.float32),
          jax.ShapeDtypeStruct((NP, HD), jnp.float32),
      ],
  )(degt, x)


def _mlp_body(acc_ref, g0_ref, g1_ref, dinv_ref, w1_ref, b1_ref, w2_ref,
              wc_ref, b2_ref, bc_ref, gz_ref, c0_ref):
  q = dinv_ref[...] * jnp.concatenate(
      [acc_ref[0] + g0_ref[...], acc_ref[1] + g1_ref[...]], axis=1)
  y = jnp.maximum(
      jnp.dot(q, w1_ref[...], preferred_element_type=jnp.float32)
      + b1_ref[...], 0.0)
  u = jnp.dot(w2_ref[...], wc_ref[...], preferred_element_type=jnp.float32)
  z = jnp.dot(y, u, preferred_element_type=jnp.float32)
  gz_ref[...] = dinv_ref[...] * z
  c0_ref[...] = jnp.dot(b2_ref[...], wc_ref[...],
                        preferred_element_type=jnp.float32) + bc_ref[...]


def _tc_mlp(acc, g0, g1, dinv, W1, b1r, W2, Wc, b2r, bc2):
  return pl.pallas_call(
      _mlp_body,
      grid=(N // _RB,),
      in_specs=[
          pl.BlockSpec((NC, _RB, HD), lambda i: (0, i, 0)),
          pl.BlockSpec((_RB, HD), lambda i: (i, 0)),
          pl.BlockSpec((_RB, HD), lambda i: (i, 0)),
          pl.BlockSpec((_RB, 1), lambda i: (i, 0)),
          pl.BlockSpec((H, H), lambda i: (0, 0)),
          pl.BlockSpec((1, H), lambda i: (0, 0)),
          pl.BlockSpec((H, H), lambda i: (0, 0)),
          pl.BlockSpec((H, 1), lambda i: (0, 0)),
          pl.BlockSpec((1, H), lambda i: (0, 0)),
          pl.BlockSpec((1, 1), lambda i: (0, 0)),
      ],
      out_specs=[
          pl.BlockSpec((_RB, 1), lambda i: (i, 0)),
          pl.BlockSpec((1, 1), lambda i: (0, 0)),
      ],
      out_shape=[
          jax.ShapeDtypeStruct((NP, 1), jnp.float32),
          jax.ShapeDtypeStruct((1, 1), jnp.float32),
      ],
  )(acc, g0, g1, dinv, W1, b1r, W2, Wc, b2r, bc2)


def kernel(x, edge_index, batch, W1, b1, W2, b2, Wc, bc):
  src = edge_index[0]
  dst = edge_index[1]
  src2 = src.reshape(NS, NB_W, EK)
  dst2 = dst.reshape(NS, NB_W, EK)
  src_f = src.reshape(NW, EPT_S)
  dst3 = dst.reshape(NW, NB_S, EK)
  batch_pad = jnp.concatenate([batch, jnp.full((NP - N,), G, jnp.int32)])
  batch3 = batch_pad.reshape(NS, NPS, EK)
  zeros_n = jnp.zeros((NP,), jnp.float32)
  zeros_nh = jnp.zeros((NP, HD), jnp.float32)
  ones_k = jnp.ones((EK,), jnp.float32)

  deg_p, cnt_p = _sc_deg(dst3, batch3, zeros_n, ones_k)
  dinv, g0, g1 = _tc_scale(deg_p.T, x)
  acc = _sc_wide(src2, dst2, g0, g1, zeros_nh)
  gz, c0 = _tc_mlp(acc, g0, g1, dinv, W1, b1.reshape(1, H), W2, Wc,
                   b2.reshape(1, H), bc.reshape(1, 1))
  num_p = _sc_scalar_pool(src_f, dst3, gz.reshape(NP), dinv.reshape(NP),
                          batch3, zeros_n)
  num = num_p[0, :G] + num_p[1, :G]
  cnt = cnt_p[0, :G] + cnt_p[1, :G]
  out = num / jnp.maximum(cnt, 1.0) + jnp.where(cnt > 0, c0[0, 0], bc[0])
  return out.reshape(G, 1)
